# Initial kernel scaffold; baseline (speedup 1.0000x reference)
import jax
import jax.numpy as jnp
from jax.experimental import pallas as pl


def kernel(x):
    raise NotImplementedError("write your pallas kernel here")


# speedup vs baseline: 1.7491x; 1.7491x over previous
import jax
import jax.numpy as jnp
from jax.experimental import pallas as pl

_ORDER = (0, 1, 2, 3, 4, 5, 6, 7, 8, 9, 10, 11, 12, 13, 14, 15, 16)
_IS_IDENTITY = _ORDER == tuple(range(len(_ORDER)))


def _reorder_block(x_ref, o_ref):
    if _IS_IDENTITY:
        o_ref[...] = x_ref[...]
    else:
        for j, s in enumerate(_ORDER):
            o_ref[:, :, j, :] = x_ref[:, :, s, :]


def kernel(x):
    b, n, j, d = x.shape
    grid = (b,)
    return pl.pallas_call(
        _reorder_block,
        grid=grid,
        in_specs=[pl.BlockSpec((1, n, j, d), lambda i: (i, 0, 0, 0))],
        out_specs=pl.BlockSpec((1, n, j, d), lambda i: (i, 0, 0, 0)),
        out_shape=jax.ShapeDtypeStruct((b, n, j, d), x.dtype),
    )(x)
